# baseline (speedup 1.0000x reference)
import functools

import jax
import jax.numpy as jnp
from jax import lax
from jax.experimental import pallas as pl
from jax.experimental.pallas import tpu as pltpu
from jax.experimental.pallas import tpu_sc as plsc

_LANES = 16
_NW = 32


def _wid():
    return lax.axis_index("s") * 2 + lax.axis_index("c")


@functools.lru_cache(maxsize=None)
def _build_invert(K, N, SPLIT):
    C = 2000
    assert N % C == 0 and C % _LANES == 0 and N % _LANES == 0
    mesh = plsc.VectorSubcoreMesh(core_axis_name="c", subcore_axis_name="s")

    def body(outf, inf, valf, g, buf, oc, ic, vc):
        w = _wid()

        @pl.when(w < K)
        def _():
            in_a = w < SPLIT
            lbase = jnp.where(in_a, w, w - SPLIT) * N
            sbase = jnp.where(in_a, SPLIT, K - SPLIT) * N
            lanes = jnp.arange(_LANES, dtype=jnp.int32)

            def init(i, carry):
                buf[pl.ds(i * _LANES, _LANES)] = lanes + (sbase + i * _LANES)
                return carry

            lax.fori_loop(0, N // _LANES, init, 0)
            base = w * N

            def chunk(c, carry):
                off = base + c * C
                pltpu.sync_copy(outf.at[pl.ds(off, C)], oc)
                pltpu.sync_copy(inf.at[pl.ds(off, C)], ic)
                pltpu.sync_copy(valf.at[pl.ds(off, C)], vc)

                def scat(j, carry2):
                    s = pl.ds(j * _LANES, _LANES)
                    plsc.store_scatter(buf, [oc[s]], ic[s] + lbase,
                                       mask=vc[s] > 0.0)
                    return carry2

                lax.fori_loop(0, C // _LANES, scat, 0)
                return carry

            lax.fori_loop(0, N // C, chunk, 0)
            pltpu.sync_copy(buf, g.at[pl.ds(base, N)])

    return pl.kernel(
        body,
        out_type=jax.ShapeDtypeStruct((K * N,), jnp.int32),
        mesh=mesh,
        compiler_params=pltpu.CompilerParams(needs_layout_passes=False),
        scratch_types=[
            pltpu.VMEM((N,), jnp.int32),
            pltpu.VMEM((C,), jnp.int32),
            pltpu.VMEM((C,), jnp.int32),
            pltpu.VMEM((C,), jnp.float32),
        ],
    )


@functools.lru_cache(maxsize=None)
def _build_gather_accum(KT, N, CH, GOFF, chain):
    R = 392
    NCH = -(-N // R)
    assert NCH % _NW == 0, (N, R, NCH)
    PER = NCH // _NW
    assert N >= R and N % 8 == 0 and R % 8 == 0 and (N - R) % 8 == 0
    mesh = plsc.VectorSubcoreMesh(core_axis_name="c", subcore_axis_name="s")

    def body(*refs):
        if chain:
            z, g, hprev, h, idxv, acc, semi, semg, semw = refs
        else:
            z, g, h, idxv, acc, semi, semg, semw = refs
        w = _wid()

        def start_of(it):
            return jnp.minimum((w + it * _NW) * R, N - R)

        def idx_copy(it, buf, kk):
            start = start_of(it)
            return pltpu.make_async_copy(
                g.at[pl.ds((GOFF + kk) * N + start, R)],
                idxv.at[pl.ds(buf * KT * R + kk * R, R)], semi,
            )

        def fire_idx(it, buf):
            lax.fori_loop(0, KT, lambda kk, c: [idx_copy(it, buf, kk).start(), c][1], 0)

        def drain_idx(it, buf):
            lax.fori_loop(0, KT, lambda kk, c: [idx_copy(it, buf, kk).wait(), c][1], 0)

        fire_idx(0, 0)
        for it in range(PER):
            buf = it % 2
            start = start_of(it)
            a = acc.at[pl.ds(buf * R, R)]
            drain_idx(it, buf)
            if it + 1 < PER:
                fire_idx(it + 1, 1 - buf)
            if it >= 2:
                pltpu.make_async_copy(a, h.at[pl.ds(start_of(it - 2), R)],
                                      semw).wait()
            if chain:
                cp0 = pltpu.make_async_copy(hprev.at[pl.ds(start, R)], a, semg)
                k_lo = 0
            else:
                cp0 = pltpu.make_async_copy(
                    z.at[idxv.at[pl.ds(buf * KT * R, R)]], a, semg)
                k_lo = 1
            cp0.start()
            cp0.wait()

            def fire_k(kk, c2):
                pltpu.async_copy(
                    z.at[idxv.at[pl.ds(buf * KT * R + kk * R, R)]], a, semg,
                    add=True)
                return c2

            lax.fori_loop(k_lo, KT, fire_k, 0)

            def drain_k(kk, c2):
                pltpu.make_async_copy(
                    z.at[idxv.at[pl.ds(buf * KT * R + kk * R, R)]], a,
                    semg).wait()
                return c2

            lax.fori_loop(k_lo, KT, drain_k, 0)
            pltpu.make_async_copy(a, h.at[pl.ds(start, R)], semw).start()

        for it in range(max(PER - 2, 0), PER):
            buf = it % 2
            pltpu.make_async_copy(acc.at[pl.ds(buf * R, R)],
                                  h.at[pl.ds(start_of(it), R)], semw).wait()

    return pl.kernel(
        body,
        out_type=jax.ShapeDtypeStruct((N, CH), jnp.float32),
        mesh=mesh,
        scratch_types=[
            pltpu.VMEM((2 * KT * R,), jnp.int32),
            pltpu.VMEM((2 * R, CH), jnp.float32),
            pltpu.SemaphoreType.DMA,
            pltpu.SemaphoreType.DMA,
            pltpu.SemaphoreType.DMA,
        ],
    )


def _tap_matmul(x, W, a=None, b=None):
    N, CIN = x.shape
    KP, _, COUT = W.shape
    TILE = 2000
    G = max(d for d in range(1, 15) if KP % d == 0)
    assert N % TILE == 0 and KP % G == 0
    T = N // TILE
    affine = a is not None

    def body(*refs):
        if affine:
            x_ref, w_ref, a_ref, b_ref, o_ref = refs
        else:
            x_ref, w_ref, o_ref = refs
        xv = x_ref[...]
        if affine:
            xv = jnp.maximum(xv * a_ref[...] + b_ref[...], 0.0)
        xv = xv.astype(jnp.bfloat16)
        for g in range(G):
            o_ref[g] = jnp.dot(xv, w_ref[g].astype(jnp.bfloat16),
                               preferred_element_type=jnp.float32)

    in_specs = [
        pl.BlockSpec((TILE, CIN), lambda t, j: (t, 0)),
        pl.BlockSpec((G, CIN, COUT), lambda t, j: (j, 0, 0)),
    ]
    args = [x, W]
    if affine:
        in_specs += [
            pl.BlockSpec((1, CIN), lambda t, j: (0, 0)),
            pl.BlockSpec((1, CIN), lambda t, j: (0, 0)),
        ]
        args += [a, b]
    z3 = pl.pallas_call(
        body,
        grid=(T, KP // G),
        in_specs=in_specs,
        out_specs=pl.BlockSpec((G, TILE, COUT), lambda t, j: (j, t, 0)),
        out_shape=jax.ShapeDtypeStruct((KP, N, COUT), jnp.float32),
    )(*args)
    return z3.reshape(KP * N, COUT)


def _col_stats(x):
    N, CH = x.shape
    TILE = 2000
    T = N // TILE

    def body(x_ref, o_ref):
        @pl.when(pl.program_id(0) == 0)
        def _():
            o_ref[...] = jnp.zeros_like(o_ref)

        xv = x_ref[...]
        o_ref[0:1, :] += jnp.sum(xv, axis=0, keepdims=True)
        o_ref[1:2, :] += jnp.sum(xv * xv, axis=0, keepdims=True)

    return pl.pallas_call(
        body,
        grid=(T,),
        in_specs=[pl.BlockSpec((TILE, CH), lambda t: (t, 0))],
        out_specs=pl.BlockSpec((8, CH), lambda t: (0, 0)),
        out_shape=jax.ShapeDtypeStruct((8, CH), jnp.float32),
    )(x)


def _residual_epilogue(h, a, b, res):
    N, CH = h.shape
    TILE = 2000
    T = N // TILE

    def body(h_ref, a_ref, b_ref, r_ref, o_ref):
        o_ref[...] = jnp.maximum(
            h_ref[...] * a_ref[...] + b_ref[...] + r_ref[...], 0.0)

    return pl.pallas_call(
        body,
        grid=(T,),
        in_specs=[
            pl.BlockSpec((TILE, CH), lambda t: (t, 0)),
            pl.BlockSpec((1, CH), lambda t: (0, 0)),
            pl.BlockSpec((1, CH), lambda t: (0, 0)),
            pl.BlockSpec((TILE, CH), lambda t: (t, 0)),
        ],
        out_specs=pl.BlockSpec((TILE, CH), lambda t: (t, 0)),
        out_shape=jax.ShapeDtypeStruct((N, CH), jnp.float32),
    )(h, a, b, res)


def _bn_affine(stats, gamma, beta, n, eps=1e-5):
    mu = stats[0] / n
    var = stats[1] / n - mu * mu
    a = gamma * lax.rsqrt(var + eps)
    b = beta - mu * a
    return a.reshape(1, -1), b.reshape(1, -1)


def kernel(feats, W1, gamma1, beta1, W2, gamma2, beta2, in_idx, out_idx, valid):
    N, _ = feats.shape
    K = W1.shape[0]

    in_f = in_idx.reshape(-1).astype(jnp.int32)
    out_f = out_idx.reshape(-1).astype(jnp.int32)
    val_f = valid.reshape(-1).astype(jnp.float32)

    CH = feats.shape[1]
    pad = jnp.zeros((1,) + W1.shape[1:], W1.dtype)
    W1p = jnp.concatenate([W1, pad], axis=0)
    W2p = jnp.concatenate([W2, pad], axis=0)

    g = _build_invert(K, N, K)(out_f, in_f, val_f)
    gacc = _build_gather_accum(K, N, CH, 0, False)

    h1 = gacc(_tap_matmul(feats, W1p), g)
    a1, b1 = _bn_affine(_col_stats(h1), gamma1, beta1, N)

    h2 = gacc(_tap_matmul(h1, W2p, a1, b1), g)
    a2, b2 = _bn_affine(_col_stats(h2), gamma2, beta2, N)

    return _residual_epilogue(h2, a2, b2, feats)


# speedup vs baseline: 1.0200x; 1.0200x over previous
import functools

import jax
import jax.numpy as jnp
from jax import lax
from jax.experimental import pallas as pl
from jax.experimental.pallas import tpu as pltpu
from jax.experimental.pallas import tpu_sc as plsc

_LANES = 16
_NW = 32


def _wid():
    return lax.axis_index("s") * 2 + lax.axis_index("c")


@functools.lru_cache(maxsize=None)
def _build_invert(K, N, SPLIT):
    C = 2000
    assert N % C == 0 and C % _LANES == 0 and N % _LANES == 0
    mesh = plsc.VectorSubcoreMesh(core_axis_name="c", subcore_axis_name="s")

    def body(outf, inf, valf, g, buf, oc, ic, vc):
        w = _wid()

        @pl.when(w < K)
        def _():
            in_a = w < SPLIT
            lbase = jnp.where(in_a, w, w - SPLIT) * N
            sbase = jnp.where(in_a, SPLIT, K - SPLIT) * N
            lanes = jnp.arange(_LANES, dtype=jnp.int32)

            def init(i, carry):
                buf[pl.ds(i * _LANES, _LANES)] = lanes + (sbase + i * _LANES)
                return carry

            lax.fori_loop(0, N // _LANES, init, 0)
            base = w * N

            def chunk(c, carry):
                off = base + c * C
                pltpu.sync_copy(outf.at[pl.ds(off, C)], oc)
                pltpu.sync_copy(inf.at[pl.ds(off, C)], ic)
                pltpu.sync_copy(valf.at[pl.ds(off, C)], vc)

                def scat(j, carry2):
                    s = pl.ds(j * _LANES, _LANES)
                    plsc.store_scatter(buf, [oc[s]], ic[s] + lbase,
                                       mask=vc[s] > 0.0)
                    return carry2

                lax.fori_loop(0, C // _LANES, scat, 0)
                return carry

            lax.fori_loop(0, N // C, chunk, 0)
            pltpu.sync_copy(buf, g.at[pl.ds(base, N)])

    return pl.kernel(
        body,
        out_type=jax.ShapeDtypeStruct((K * N,), jnp.int32),
        mesh=mesh,
        compiler_params=pltpu.CompilerParams(needs_layout_passes=False),
        scratch_types=[
            pltpu.VMEM((N,), jnp.int32),
            pltpu.VMEM((C,), jnp.int32),
            pltpu.VMEM((C,), jnp.int32),
            pltpu.VMEM((C,), jnp.float32),
        ],
    )


@functools.lru_cache(maxsize=None)
def _build_gather_accum(KT, N, CH, GOFF, chain):
    R = 392
    NCH = -(-N // R)
    assert NCH % _NW == 0, (N, R, NCH)
    PER = NCH // _NW
    assert N >= R and N % 8 == 0 and R % 8 == 0 and (N - R) % 8 == 0
    mesh = plsc.VectorSubcoreMesh(core_axis_name="c", subcore_axis_name="s")

    def body(*refs):
        if chain:
            z, g, hprev, h, idxv, acc, semi, semg, semw = refs
        else:
            z, g, h, idxv, acc, semi, semg, semw = refs
        w = _wid()

        def start_of(it):
            return jnp.minimum((w + it * _NW) * R, N - R)

        def idx_copy(it, buf, kk):
            start = start_of(it)
            return pltpu.make_async_copy(
                g.at[pl.ds((GOFF + kk) * N + start, R)],
                idxv.at[pl.ds(buf * KT * R + kk * R, R)], semi,
            )

        def fire_idx(it, buf):
            lax.fori_loop(0, KT, lambda kk, c: [idx_copy(it, buf, kk).start(), c][1], 0)

        def drain_idx(it, buf):
            lax.fori_loop(0, KT, lambda kk, c: [idx_copy(it, buf, kk).wait(), c][1], 0)

        fire_idx(0, 0)
        for it in range(PER):
            buf = it % 2
            start = start_of(it)
            a = acc.at[pl.ds(buf * R, R)]
            drain_idx(it, buf)
            if it + 1 < PER:
                fire_idx(it + 1, 1 - buf)
            if it >= 2:
                pltpu.make_async_copy(a, h.at[pl.ds(start_of(it - 2), R)],
                                      semw).wait()
            if chain:
                cp0 = pltpu.make_async_copy(hprev.at[pl.ds(start, R)], a, semg)
                k_lo = 0
            else:
                cp0 = pltpu.make_async_copy(
                    z.at[idxv.at[pl.ds(buf * KT * R, R)]], a, semg)
                k_lo = 1
            cp0.start()
            cp0.wait()

            def fire_k(kk, c2):
                pltpu.async_copy(
                    z.at[idxv.at[pl.ds(buf * KT * R + kk * R, R)]], a, semg,
                    add=True)
                return c2

            lax.fori_loop(k_lo, KT, fire_k, 0)

            def drain_k(kk, c2):
                pltpu.make_async_copy(
                    z.at[idxv.at[pl.ds(buf * KT * R + kk * R, R)]], a,
                    semg).wait()
                return c2

            lax.fori_loop(k_lo, KT, drain_k, 0)
            pltpu.make_async_copy(a, h.at[pl.ds(start, R)], semw).start()

        for it in range(max(PER - 2, 0), PER):
            buf = it % 2
            pltpu.make_async_copy(acc.at[pl.ds(buf * R, R)],
                                  h.at[pl.ds(start_of(it), R)], semw).wait()

    return pl.kernel(
        body,
        out_type=jax.ShapeDtypeStruct((N, CH), jnp.float32),
        mesh=mesh,
        scratch_types=[
            pltpu.VMEM((2 * KT * R,), jnp.int32),
            pltpu.VMEM((2 * R, CH), jnp.float32),
            pltpu.SemaphoreType.DMA,
            pltpu.SemaphoreType.DMA,
            pltpu.SemaphoreType.DMA,
        ],
    )


def _tap_matmul(x, W, a=None, b=None):
    N, CIN = x.shape
    KP, _, COUT = W.shape
    TILE = 5000
    G = max(d for d in range(1, 8) if KP % d == 0)
    assert N % TILE == 0 and KP % G == 0
    T = N // TILE
    affine = a is not None

    def body(*refs):
        if affine:
            x_ref, w_ref, a_ref, b_ref, o_ref = refs
        else:
            x_ref, w_ref, o_ref = refs
        xv = x_ref[...]
        if affine:
            xv = jnp.maximum(xv * a_ref[...] + b_ref[...], 0.0)
        for g in range(G):
            o_ref[g] = jnp.dot(xv, w_ref[g], preferred_element_type=jnp.float32)

    in_specs = [
        pl.BlockSpec((TILE, CIN), lambda t, j: (t, 0)),
        pl.BlockSpec((G, CIN, COUT), lambda t, j: (j, 0, 0)),
    ]
    args = [x, W]
    if affine:
        in_specs += [
            pl.BlockSpec((1, CIN), lambda t, j: (0, 0)),
            pl.BlockSpec((1, CIN), lambda t, j: (0, 0)),
        ]
        args += [a, b]
    z3 = pl.pallas_call(
        body,
        grid=(T, KP // G),
        in_specs=in_specs,
        out_specs=pl.BlockSpec((G, TILE, COUT), lambda t, j: (j, t, 0)),
        out_shape=jax.ShapeDtypeStruct((KP, N, COUT), jnp.float32),
    )(*args)
    return z3.reshape(KP * N, COUT)


def _col_stats(x):
    N, CH = x.shape
    TILE = 2000
    T = N // TILE

    def body(x_ref, o_ref):
        @pl.when(pl.program_id(0) == 0)
        def _():
            o_ref[...] = jnp.zeros_like(o_ref)

        xv = x_ref[...]
        o_ref[0:1, :] += jnp.sum(xv, axis=0, keepdims=True)
        o_ref[1:2, :] += jnp.sum(xv * xv, axis=0, keepdims=True)

    return pl.pallas_call(
        body,
        grid=(T,),
        in_specs=[pl.BlockSpec((TILE, CH), lambda t: (t, 0))],
        out_specs=pl.BlockSpec((8, CH), lambda t: (0, 0)),
        out_shape=jax.ShapeDtypeStruct((8, CH), jnp.float32),
    )(x)


def _residual_epilogue(h, a, b, res):
    N, CH = h.shape
    TILE = 2000
    T = N // TILE

    def body(h_ref, a_ref, b_ref, r_ref, o_ref):
        o_ref[...] = jnp.maximum(
            h_ref[...] * a_ref[...] + b_ref[...] + r_ref[...], 0.0)

    return pl.pallas_call(
        body,
        grid=(T,),
        in_specs=[
            pl.BlockSpec((TILE, CH), lambda t: (t, 0)),
            pl.BlockSpec((1, CH), lambda t: (0, 0)),
            pl.BlockSpec((1, CH), lambda t: (0, 0)),
            pl.BlockSpec((TILE, CH), lambda t: (t, 0)),
        ],
        out_specs=pl.BlockSpec((TILE, CH), lambda t: (t, 0)),
        out_shape=jax.ShapeDtypeStruct((N, CH), jnp.float32),
    )(h, a, b, res)


def _bn_affine(stats, gamma, beta, n, eps=1e-5):
    mu = stats[0] / n
    var = stats[1] / n - mu * mu
    a = gamma * lax.rsqrt(var + eps)
    b = beta - mu * a
    return a.reshape(1, -1), b.reshape(1, -1)


def kernel(feats, W1, gamma1, beta1, W2, gamma2, beta2, in_idx, out_idx, valid):
    N, _ = feats.shape
    K = W1.shape[0]

    in_f = in_idx.reshape(-1).astype(jnp.int32)
    out_f = out_idx.reshape(-1).astype(jnp.int32)
    val_f = valid.reshape(-1).astype(jnp.float32)

    CH = feats.shape[1]
    pad = jnp.zeros((1,) + W1.shape[1:], W1.dtype)
    W1p = jnp.concatenate([W1, pad], axis=0)
    W2p = jnp.concatenate([W2, pad], axis=0)

    g = _build_invert(K, N, K)(out_f, in_f, val_f)
    gacc = _build_gather_accum(K, N, CH, 0, False)

    h1 = gacc(_tap_matmul(feats, W1p), g)
    a1, b1 = _bn_affine(_col_stats(h1), gamma1, beta1, N)

    h2 = gacc(_tap_matmul(h1, W2p, a1, b1), g)
    a2, b2 = _bn_affine(_col_stats(h2), gamma2, beta2, N)

    return _residual_epilogue(h2, a2, b2, feats)


# speedup vs baseline: 1.0370x; 1.0167x over previous
import functools

import jax
import jax.numpy as jnp
from jax import lax
from jax.experimental import pallas as pl
from jax.experimental.pallas import tpu as pltpu
from jax.experimental.pallas import tpu_sc as plsc

_LANES = 16
_NW = 32


def _wid():
    return lax.axis_index("s") * 2 + lax.axis_index("c")


@functools.lru_cache(maxsize=None)
def _build_invert(K, N, SPLIT):
    C = 2000
    assert N % C == 0 and C % _LANES == 0 and N % _LANES == 0
    mesh = plsc.VectorSubcoreMesh(core_axis_name="c", subcore_axis_name="s")

    def body(outf, inf, valf, g, buf, oc, ic, vc):
        w = _wid()

        @pl.when(w < K)
        def _():
            in_a = w < SPLIT
            lbase = jnp.where(in_a, w, w - SPLIT) * N
            sbase = jnp.where(in_a, SPLIT, K - SPLIT) * N
            lanes = jnp.arange(_LANES, dtype=jnp.int32)

            def init(i, carry):
                buf[pl.ds(i * _LANES, _LANES)] = lanes + (sbase + i * _LANES)
                return carry

            lax.fori_loop(0, N // _LANES, init, 0)
            base = w * N

            def chunk(c, carry):
                off = base + c * C
                pltpu.sync_copy(outf.at[pl.ds(off, C)], oc)
                pltpu.sync_copy(inf.at[pl.ds(off, C)], ic)
                pltpu.sync_copy(valf.at[pl.ds(off, C)], vc)

                def scat(j, carry2):
                    s = pl.ds(j * _LANES, _LANES)
                    plsc.store_scatter(buf, [oc[s]], ic[s] + lbase,
                                       mask=vc[s] > 0.0)
                    return carry2

                lax.fori_loop(0, C // _LANES, scat, 0)
                return carry

            lax.fori_loop(0, N // C, chunk, 0)
            pltpu.sync_copy(buf, g.at[pl.ds(base, N)])

    return pl.kernel(
        body,
        out_type=jax.ShapeDtypeStruct((K * N,), jnp.int32),
        mesh=mesh,
        compiler_params=pltpu.CompilerParams(needs_layout_passes=False),
        scratch_types=[
            pltpu.VMEM((N,), jnp.int32),
            pltpu.VMEM((C,), jnp.int32),
            pltpu.VMEM((C,), jnp.int32),
            pltpu.VMEM((C,), jnp.float32),
        ],
    )


@functools.lru_cache(maxsize=None)
def _build_gather_accum(KT, N, CH, GOFF, chain):
    R = 392
    NCH = -(-N // R)
    assert NCH % _NW == 0, (N, R, NCH)
    PER = NCH // _NW
    assert N >= R and N % 8 == 0 and R % 8 == 0 and (N - R) % 8 == 0
    mesh = plsc.VectorSubcoreMesh(core_axis_name="c", subcore_axis_name="s")

    def body(*refs):
        if chain:
            z, g, hprev, h, idxv, acc, semi, semg, semw = refs
        else:
            z, g, h, idxv, acc, semi, semg, semw = refs
        w = _wid()

        def start_of(it):
            return jnp.minimum((w + it * _NW) * R, N - R)

        def idx_copy(it, buf, kk):
            start = start_of(it)
            return pltpu.make_async_copy(
                g.at[pl.ds((GOFF + kk) * N + start, R)],
                idxv.at[pl.ds(buf * KT * R + kk * R, R)], semi,
            )

        def fire_idx(it, buf):
            lax.fori_loop(0, KT, lambda kk, c: [idx_copy(it, buf, kk).start(), c][1], 0)

        def drain_idx(it, buf):
            lax.fori_loop(0, KT, lambda kk, c: [idx_copy(it, buf, kk).wait(), c][1], 0)

        fire_idx(0, 0)
        for it in range(PER):
            buf = it % 2
            start = start_of(it)
            a = acc.at[pl.ds(buf * R, R)]
            drain_idx(it, buf)
            if it + 1 < PER:
                fire_idx(it + 1, 1 - buf)
            if it >= 2:
                pltpu.make_async_copy(a, h.at[pl.ds(start_of(it - 2), R)],
                                      semw).wait()
            if chain:
                cp0 = pltpu.make_async_copy(hprev.at[pl.ds(start, R)], a, semg)
                k_lo = 0
            else:
                cp0 = pltpu.make_async_copy(
                    z.at[idxv.at[pl.ds(buf * KT * R, R)]], a, semg)
                k_lo = 1
            cp0.start()
            cp0.wait()

            def fire_k(kk, c2):
                pltpu.async_copy(
                    z.at[idxv.at[pl.ds(buf * KT * R + kk * R, R)]], a, semg,
                    add=True)
                return c2

            lax.fori_loop(k_lo, KT, fire_k, 0)

            def drain_k(kk, c2):
                pltpu.make_async_copy(
                    z.at[idxv.at[pl.ds(buf * KT * R + kk * R, R)]], a,
                    semg).wait()
                return c2

            lax.fori_loop(k_lo, KT, drain_k, 0)
            pltpu.make_async_copy(a, h.at[pl.ds(start, R)], semw).start()

        for it in range(max(PER - 2, 0), PER):
            buf = it % 2
            pltpu.make_async_copy(acc.at[pl.ds(buf * R, R)],
                                  h.at[pl.ds(start_of(it), R)], semw).wait()

    return pl.kernel(
        body,
        out_type=jax.ShapeDtypeStruct((N, CH), jnp.float32),
        mesh=mesh,
        scratch_types=[
            pltpu.VMEM((2 * KT * R,), jnp.int32),
            pltpu.VMEM((2 * R, CH), jnp.float32),
            pltpu.SemaphoreType.DMA,
            pltpu.SemaphoreType.DMA,
            pltpu.SemaphoreType.DMA,
        ],
    )


def _tap_matmul(x, W, a=None, b=None):
    N, CIN = x.shape
    KP, _, COUT = W.shape
    TILE = 10000
    G = max(d for d in range(1, 5) if KP % d == 0)
    assert N % TILE == 0 and KP % G == 0
    T = N // TILE
    affine = a is not None

    def body(*refs):
        if affine:
            x_ref, w_ref, a_ref, b_ref, o_ref = refs
        else:
            x_ref, w_ref, o_ref = refs
        xv = x_ref[...]
        if affine:
            xv = jnp.maximum(xv * a_ref[...] + b_ref[...], 0.0)
        for g in range(G):
            o_ref[g] = jnp.dot(xv, w_ref[g], preferred_element_type=jnp.float32)

    in_specs = [
        pl.BlockSpec((TILE, CIN), lambda t, j: (t, 0)),
        pl.BlockSpec((G, CIN, COUT), lambda t, j: (j, 0, 0)),
    ]
    args = [x, W]
    if affine:
        in_specs += [
            pl.BlockSpec((1, CIN), lambda t, j: (0, 0)),
            pl.BlockSpec((1, CIN), lambda t, j: (0, 0)),
        ]
        args += [a, b]
    z3 = pl.pallas_call(
        body,
        grid=(T, KP // G),
        in_specs=in_specs,
        out_specs=pl.BlockSpec((G, TILE, COUT), lambda t, j: (j, t, 0)),
        out_shape=jax.ShapeDtypeStruct((KP, N, COUT), jnp.float32),
    )(*args)
    return z3.reshape(KP * N, COUT)


def _col_stats(x):
    N, CH = x.shape
    TILE = 2000
    T = N // TILE

    def body(x_ref, o_ref):
        @pl.when(pl.program_id(0) == 0)
        def _():
            o_ref[...] = jnp.zeros_like(o_ref)

        xv = x_ref[...]
        o_ref[0:1, :] += jnp.sum(xv, axis=0, keepdims=True)
        o_ref[1:2, :] += jnp.sum(xv * xv, axis=0, keepdims=True)

    return pl.pallas_call(
        body,
        grid=(T,),
        in_specs=[pl.BlockSpec((TILE, CH), lambda t: (t, 0))],
        out_specs=pl.BlockSpec((8, CH), lambda t: (0, 0)),
        out_shape=jax.ShapeDtypeStruct((8, CH), jnp.float32),
    )(x)


def _residual_epilogue(h, a, b, res):
    N, CH = h.shape
    TILE = 2000
    T = N // TILE

    def body(h_ref, a_ref, b_ref, r_ref, o_ref):
        o_ref[...] = jnp.maximum(
            h_ref[...] * a_ref[...] + b_ref[...] + r_ref[...], 0.0)

    return pl.pallas_call(
        body,
        grid=(T,),
        in_specs=[
            pl.BlockSpec((TILE, CH), lambda t: (t, 0)),
            pl.BlockSpec((1, CH), lambda t: (0, 0)),
            pl.BlockSpec((1, CH), lambda t: (0, 0)),
            pl.BlockSpec((TILE, CH), lambda t: (t, 0)),
        ],
        out_specs=pl.BlockSpec((TILE, CH), lambda t: (t, 0)),
        out_shape=jax.ShapeDtypeStruct((N, CH), jnp.float32),
    )(h, a, b, res)


def _bn_affine(stats, gamma, beta, n, eps=1e-5):
    mu = stats[0] / n
    var = stats[1] / n - mu * mu
    a = gamma * lax.rsqrt(var + eps)
    b = beta - mu * a
    return a.reshape(1, -1), b.reshape(1, -1)


def kernel(feats, W1, gamma1, beta1, W2, gamma2, beta2, in_idx, out_idx, valid):
    N, _ = feats.shape
    K = W1.shape[0]

    in_f = in_idx.reshape(-1).astype(jnp.int32)
    out_f = out_idx.reshape(-1).astype(jnp.int32)
    val_f = valid.reshape(-1).astype(jnp.float32)

    CH = feats.shape[1]
    pad = jnp.zeros((1,) + W1.shape[1:], W1.dtype)
    W1p = jnp.concatenate([W1, pad], axis=0)
    W2p = jnp.concatenate([W2, pad], axis=0)

    g = _build_invert(K, N, K)(out_f, in_f, val_f)
    gacc = _build_gather_accum(K, N, CH, 0, False)

    h1 = gacc(_tap_matmul(feats, W1p), g)
    a1, b1 = _bn_affine(_col_stats(h1), gamma1, beta1, N)

    h2 = gacc(_tap_matmul(h1, W2p, a1, b1), g)
    a2, b2 = _bn_affine(_col_stats(h2), gamma2, beta2, N)

    return _residual_epilogue(h2, a2, b2, feats)


# speedup vs baseline: 1.0819x; 1.0433x over previous
import functools

import jax
import jax.numpy as jnp
from jax import lax
from jax.experimental import pallas as pl
from jax.experimental.pallas import tpu as pltpu
from jax.experimental.pallas import tpu_sc as plsc

_LANES = 16
_NW = 32


def _wid():
    return lax.axis_index("s") * 2 + lax.axis_index("c")


@functools.lru_cache(maxsize=None)
def _build_invert(K, N, SPLIT):
    C = 2000
    assert N % C == 0 and C % _LANES == 0 and N % _LANES == 0
    mesh = plsc.VectorSubcoreMesh(core_axis_name="c", subcore_axis_name="s")

    def body(outf, inf, valf, g, buf, oc, ic, vc):
        w = _wid()

        @pl.when(w < K)
        def _():
            in_a = w < SPLIT
            lbase = jnp.where(in_a, w, w - SPLIT) * N
            sbase = jnp.where(in_a, SPLIT, K - SPLIT) * N
            lanes = jnp.arange(_LANES, dtype=jnp.int32)

            def init(i, carry):
                buf[pl.ds(i * _LANES, _LANES)] = lanes + (sbase + i * _LANES)
                return carry

            lax.fori_loop(0, N // _LANES, init, 0)
            base = w * N

            def chunk(c, carry):
                off = base + c * C
                pltpu.sync_copy(outf.at[pl.ds(off, C)], oc)
                pltpu.sync_copy(inf.at[pl.ds(off, C)], ic)
                pltpu.sync_copy(valf.at[pl.ds(off, C)], vc)

                def scat(j, carry2):
                    s = pl.ds(j * _LANES, _LANES)
                    plsc.store_scatter(buf, [oc[s]], ic[s] + lbase,
                                       mask=vc[s] > 0.0)
                    return carry2

                lax.fori_loop(0, C // _LANES, scat, 0)
                return carry

            lax.fori_loop(0, N // C, chunk, 0)
            pltpu.sync_copy(buf, g.at[pl.ds(base, N)])

    return pl.kernel(
        body,
        out_type=jax.ShapeDtypeStruct((K * N,), jnp.int32),
        mesh=mesh,
        compiler_params=pltpu.CompilerParams(needs_layout_passes=False),
        scratch_types=[
            pltpu.VMEM((N,), jnp.int32),
            pltpu.VMEM((C,), jnp.int32),
            pltpu.VMEM((C,), jnp.int32),
            pltpu.VMEM((C,), jnp.float32),
        ],
    )


@functools.lru_cache(maxsize=None)
def _build_gather_accum(KT, N, CH, GOFF, chain):
    R = 392
    NCH = -(-N // R)
    assert NCH % _NW == 0, (N, R, NCH)
    PER = NCH // _NW
    assert N >= R and N % 8 == 0 and R % 8 == 0 and (N - R) % 8 == 0
    mesh = plsc.VectorSubcoreMesh(core_axis_name="c", subcore_axis_name="s")

    NB = CH // _LANES

    def body(*refs):
        if chain:
            z, g, hprev, h, parts, idxv, acc, st, semi, semg, semw = refs
        else:
            z, g, h, parts, idxv, acc, st, semi, semg, semw = refs
        w = _wid()

        def vpu_stats(buf, skip, carry):
            svs, qvs = carry

            def row(r, c2):
                s2, q2 = c2
                vs = [acc[buf * R + r, pl.ds(bb * _LANES, _LANES)]
                      for bb in range(NB)]
                s2 = tuple(s2[bb] + vs[bb] for bb in range(NB))
                q2 = tuple(q2[bb] + vs[bb] * vs[bb] for bb in range(NB))
                return (s2, q2)

            return lax.fori_loop(skip, R, row, (svs, qvs))

        def start_of(it):
            return jnp.minimum((w + it * _NW) * R, N - R)

        def idx_copy(it, buf, kk):
            start = start_of(it)
            return pltpu.make_async_copy(
                g.at[pl.ds((GOFF + kk) * N + start, R)],
                idxv.at[pl.ds(buf * KT * R + kk * R, R)], semi,
            )

        def fire_idx(it, buf):
            lax.fori_loop(0, KT, lambda kk, c: [idx_copy(it, buf, kk).start(), c][1], 0)

        def drain_idx(it, buf):
            lax.fori_loop(0, KT, lambda kk, c: [idx_copy(it, buf, kk).wait(), c][1], 0)

        def skip_of(it):
            return (w + it * _NW) * R - start_of(it)

        zero = jnp.zeros((_LANES,), jnp.float32)
        carry = (tuple(zero for _ in range(NB)), tuple(zero for _ in range(NB)))

        fire_idx(0, 0)
        for it in range(PER):
            buf = it % 2
            start = start_of(it)
            a = acc.at[pl.ds(buf * R, R)]
            drain_idx(it, buf)
            if it + 1 < PER:
                fire_idx(it + 1, 1 - buf)
            if it >= 2:
                pltpu.make_async_copy(a, h.at[pl.ds(start_of(it - 2), R)],
                                      semw).wait()
            if chain:
                cp0 = pltpu.make_async_copy(hprev.at[pl.ds(start, R)], a, semg)
                k_lo = 0
            else:
                cp0 = pltpu.make_async_copy(
                    z.at[idxv.at[pl.ds(buf * KT * R, R)]], a, semg)
                k_lo = 1
            cp0.start()
            cp0.wait()

            def fire_k(kk, c2):
                pltpu.async_copy(
                    z.at[idxv.at[pl.ds(buf * KT * R + kk * R, R)]], a, semg,
                    add=True)
                return c2

            lax.fori_loop(k_lo, KT, fire_k, 0)
            if it >= 1:
                carry = vpu_stats(1 - buf, skip_of(it - 1), carry)

            def drain_k(kk, c2):
                pltpu.make_async_copy(
                    z.at[idxv.at[pl.ds(buf * KT * R + kk * R, R)]], a,
                    semg).wait()
                return c2

            lax.fori_loop(k_lo, KT, drain_k, 0)
            pltpu.make_async_copy(a, h.at[pl.ds(start, R)], semw).start()

        carry = vpu_stats((PER - 1) % 2, skip_of(PER - 1), carry)
        svs, qvs = carry
        for bb in range(NB):
            st[0, pl.ds(bb * _LANES, _LANES)] = svs[bb]
            st[1, pl.ds(bb * _LANES, _LANES)] = qvs[bb]
            for rr in range(2, 8):
                st[rr, pl.ds(bb * _LANES, _LANES)] = zero
        pltpu.sync_copy(st, parts.at[w])

        for it in range(max(PER - 2, 0), PER):
            buf = it % 2
            pltpu.make_async_copy(acc.at[pl.ds(buf * R, R)],
                                  h.at[pl.ds(start_of(it), R)], semw).wait()

    return pl.kernel(
        body,
        out_type=(jax.ShapeDtypeStruct((N, CH), jnp.float32),
                  jax.ShapeDtypeStruct((_NW, 8, CH), jnp.float32)),
        mesh=mesh,
        scratch_types=[
            pltpu.VMEM((2 * KT * R,), jnp.int32),
            pltpu.VMEM((2 * R, CH), jnp.float32),
            pltpu.VMEM((8, CH), jnp.float32),
            pltpu.SemaphoreType.DMA,
            pltpu.SemaphoreType.DMA,
            pltpu.SemaphoreType.DMA,
        ],
    )


def _tap_matmul(x, W, a=None, b=None):
    N, CIN = x.shape
    KP, _, COUT = W.shape
    TILE = 10000
    G = max(d for d in range(1, 5) if KP % d == 0)
    assert N % TILE == 0 and KP % G == 0
    T = N // TILE
    affine = a is not None

    def body(*refs):
        if affine:
            x_ref, w_ref, a_ref, b_ref, o_ref = refs
        else:
            x_ref, w_ref, o_ref = refs
        xv = x_ref[...]
        if affine:
            xv = jnp.maximum(xv * a_ref[...] + b_ref[...], 0.0)
        for g in range(G):
            o_ref[g] = jnp.dot(xv, w_ref[g], preferred_element_type=jnp.float32)

    in_specs = [
        pl.BlockSpec((TILE, CIN), lambda t, j: (t, 0)),
        pl.BlockSpec((G, CIN, COUT), lambda t, j: (j, 0, 0)),
    ]
    args = [x, W]
    if affine:
        in_specs += [
            pl.BlockSpec((1, CIN), lambda t, j: (0, 0)),
            pl.BlockSpec((1, CIN), lambda t, j: (0, 0)),
        ]
        args += [a, b]
    z3 = pl.pallas_call(
        body,
        grid=(T, KP // G),
        in_specs=in_specs,
        out_specs=pl.BlockSpec((G, TILE, COUT), lambda t, j: (j, t, 0)),
        out_shape=jax.ShapeDtypeStruct((KP, N, COUT), jnp.float32),
    )(*args)
    return z3.reshape(KP * N, COUT)


def _col_stats(x):
    N, CH = x.shape
    TILE = 2000
    T = N // TILE

    def body(x_ref, o_ref):
        @pl.when(pl.program_id(0) == 0)
        def _():
            o_ref[...] = jnp.zeros_like(o_ref)

        xv = x_ref[...]
        o_ref[0:1, :] += jnp.sum(xv, axis=0, keepdims=True)
        o_ref[1:2, :] += jnp.sum(xv * xv, axis=0, keepdims=True)

    return pl.pallas_call(
        body,
        grid=(T,),
        in_specs=[pl.BlockSpec((TILE, CH), lambda t: (t, 0))],
        out_specs=pl.BlockSpec((8, CH), lambda t: (0, 0)),
        out_shape=jax.ShapeDtypeStruct((8, CH), jnp.float32),
    )(x)


def _residual_epilogue(h, a, b, res):
    N, CH = h.shape
    TILE = 2000
    T = N // TILE

    def body(h_ref, a_ref, b_ref, r_ref, o_ref):
        o_ref[...] = jnp.maximum(
            h_ref[...] * a_ref[...] + b_ref[...] + r_ref[...], 0.0)

    return pl.pallas_call(
        body,
        grid=(T,),
        in_specs=[
            pl.BlockSpec((TILE, CH), lambda t: (t, 0)),
            pl.BlockSpec((1, CH), lambda t: (0, 0)),
            pl.BlockSpec((1, CH), lambda t: (0, 0)),
            pl.BlockSpec((TILE, CH), lambda t: (t, 0)),
        ],
        out_specs=pl.BlockSpec((TILE, CH), lambda t: (t, 0)),
        out_shape=jax.ShapeDtypeStruct((N, CH), jnp.float32),
    )(h, a, b, res)


def _bn_affine(stats, gamma, beta, n, eps=1e-5):
    mu = stats[0] / n
    var = stats[1] / n - mu * mu
    a = gamma * lax.rsqrt(var + eps)
    b = beta - mu * a
    return a.reshape(1, -1), b.reshape(1, -1)


def kernel(feats, W1, gamma1, beta1, W2, gamma2, beta2, in_idx, out_idx, valid):
    N, _ = feats.shape
    K = W1.shape[0]

    in_f = in_idx.reshape(-1).astype(jnp.int32)
    out_f = out_idx.reshape(-1).astype(jnp.int32)
    val_f = valid.reshape(-1).astype(jnp.float32)

    CH = feats.shape[1]
    pad = jnp.zeros((1,) + W1.shape[1:], W1.dtype)
    W1p = jnp.concatenate([W1, pad], axis=0)
    W2p = jnp.concatenate([W2, pad], axis=0)

    g = _build_invert(K, N, K)(out_f, in_f, val_f)
    gacc = _build_gather_accum(K, N, CH, 0, False)

    h1, p1 = gacc(_tap_matmul(feats, W1p), g)
    a1, b1 = _bn_affine(jnp.sum(p1, axis=0), gamma1, beta1, N)

    h2, p2 = gacc(_tap_matmul(h1, W2p, a1, b1), g)
    a2, b2 = _bn_affine(jnp.sum(p2, axis=0), gamma2, beta2, N)

    return _residual_epilogue(h2, a2, b2, feats)


# speedup vs baseline: 1.0828x; 1.0008x over previous
import functools

import jax
import jax.numpy as jnp
from jax import lax
from jax.experimental import pallas as pl
from jax.experimental.pallas import tpu as pltpu
from jax.experimental.pallas import tpu_sc as plsc

_LANES = 16
_NW = 32


def _wid():
    return lax.axis_index("s") * 2 + lax.axis_index("c")


@functools.lru_cache(maxsize=None)
def _build_invert(K, N, SPLIT):
    C = 2000
    assert N % C == 0 and C % _LANES == 0 and N % _LANES == 0
    mesh = plsc.VectorSubcoreMesh(core_axis_name="c", subcore_axis_name="s")

    def body(outf, inf, valf, g, buf, oc, ic, vc):
        w = _wid()

        @pl.when(w < K)
        def _():
            in_a = w < SPLIT
            lbase = jnp.where(in_a, w, w - SPLIT) * N
            sbase = jnp.where(in_a, SPLIT, K - SPLIT) * N
            lanes = jnp.arange(_LANES, dtype=jnp.int32)

            def init(i, carry):
                buf[pl.ds(i * _LANES, _LANES)] = lanes + (sbase + i * _LANES)
                return carry

            lax.fori_loop(0, N // _LANES, init, 0)
            base = w * N

            def chunk(c, carry):
                off = base + c * C
                pltpu.sync_copy(outf.at[pl.ds(off, C)], oc)
                pltpu.sync_copy(inf.at[pl.ds(off, C)], ic)
                pltpu.sync_copy(valf.at[pl.ds(off, C)], vc)

                def scat(j, carry2):
                    s = pl.ds(j * _LANES, _LANES)
                    plsc.store_scatter(buf, [oc[s]], ic[s] + lbase,
                                       mask=vc[s] > 0.0)
                    return carry2

                lax.fori_loop(0, C // _LANES, scat, 0)
                return carry

            lax.fori_loop(0, N // C, chunk, 0)
            pltpu.sync_copy(buf, g.at[pl.ds(base, N)])

    return pl.kernel(
        body,
        out_type=jax.ShapeDtypeStruct((K * N,), jnp.int32),
        mesh=mesh,
        compiler_params=pltpu.CompilerParams(needs_layout_passes=False),
        scratch_types=[
            pltpu.VMEM((N,), jnp.int32),
            pltpu.VMEM((C,), jnp.int32),
            pltpu.VMEM((C,), jnp.int32),
            pltpu.VMEM((C,), jnp.float32),
        ],
    )


@functools.lru_cache(maxsize=None)
def _build_gather_accum(KT, N, CH, GOFF, chain):
    R = 392
    NCH = -(-N // R)
    assert NCH % _NW == 0, (N, R, NCH)
    PER = NCH // _NW
    assert N >= R and N % 8 == 0 and R % 8 == 0 and (N - R) % 8 == 0
    mesh = plsc.VectorSubcoreMesh(core_axis_name="c", subcore_axis_name="s")

    NB = CH // _LANES

    def body(*refs):
        if chain:
            z, g, hprev, h, parts, idxv, acc, st, semi, semg, semw = refs
        else:
            z, g, h, parts, idxv, acc, st, semi, semg, semw = refs
        w = _wid()

        def vpu_stats(buf, skip, carry):
            svs, qvs = carry

            def row(r, c2):
                s2, q2 = c2
                vs = [acc[buf * R + r, pl.ds(bb * _LANES, _LANES)]
                      for bb in range(NB)]
                s2 = tuple(s2[bb] + vs[bb] for bb in range(NB))
                q2 = tuple(q2[bb] + vs[bb] * vs[bb] for bb in range(NB))
                return (s2, q2)

            return lax.fori_loop(skip, R, row, (svs, qvs))

        def start_of(it):
            return jnp.minimum((w + it * _NW) * R, N - R)

        def idx_copy(it, buf, kk):
            start = start_of(it)
            return pltpu.make_async_copy(
                g.at[pl.ds((GOFF + kk) * N + start, R)],
                idxv.at[pl.ds(buf * KT * R + kk * R, R)], semi,
            )

        def fire_idx(it, buf):
            lax.fori_loop(0, KT, lambda kk, c: [idx_copy(it, buf, kk).start(), c][1], 0)

        def drain_idx(it, buf):
            lax.fori_loop(0, KT, lambda kk, c: [idx_copy(it, buf, kk).wait(), c][1], 0)

        def skip_of(it):
            return (w + it * _NW) * R - start_of(it)

        zero = jnp.zeros((_LANES,), jnp.float32)
        carry = (tuple(zero for _ in range(NB)), tuple(zero for _ in range(NB)))

        fire_idx(0, 0)
        for it in range(PER):
            buf = it % 2
            start = start_of(it)
            a = acc.at[pl.ds(buf * R, R)]
            drain_idx(it, buf)
            if it + 1 < PER:
                fire_idx(it + 1, 1 - buf)
            if it >= 2:
                pltpu.make_async_copy(a, h.at[pl.ds(start_of(it - 2), R)],
                                      semw).wait()
            if chain:
                cp0 = pltpu.make_async_copy(hprev.at[pl.ds(start, R)], a, semg)
                k_lo = 0
            else:
                cp0 = pltpu.make_async_copy(
                    z.at[idxv.at[pl.ds(buf * KT * R, R)]], a, semg)
                k_lo = 1
            cp0.start()
            cp0.wait()

            def fire_k(kk, c2):
                pltpu.async_copy(
                    z.at[idxv.at[pl.ds(buf * KT * R + kk * R, R)]], a, semg,
                    add=True)
                return c2

            lax.fori_loop(k_lo, KT, fire_k, 0)
            if it >= 1:
                carry = vpu_stats(1 - buf, skip_of(it - 1), carry)

            def drain_k(kk, c2):
                pltpu.make_async_copy(
                    z.at[idxv.at[pl.ds(buf * KT * R + kk * R, R)]], a,
                    semg).wait()
                return c2

            lax.fori_loop(k_lo, KT, drain_k, 0)
            pltpu.make_async_copy(a, h.at[pl.ds(start, R)], semw).start()

        carry = vpu_stats((PER - 1) % 2, skip_of(PER - 1), carry)
        svs, qvs = carry
        for bb in range(NB):
            st[0, pl.ds(bb * _LANES, _LANES)] = svs[bb]
            st[1, pl.ds(bb * _LANES, _LANES)] = qvs[bb]
            for rr in range(2, 8):
                st[rr, pl.ds(bb * _LANES, _LANES)] = zero
        pltpu.sync_copy(st, parts.at[w])

        for it in range(max(PER - 2, 0), PER):
            buf = it % 2
            pltpu.make_async_copy(acc.at[pl.ds(buf * R, R)],
                                  h.at[pl.ds(start_of(it), R)], semw).wait()

    return pl.kernel(
        body,
        out_type=(jax.ShapeDtypeStruct((N, CH), jnp.float32),
                  jax.ShapeDtypeStruct((_NW, 8, CH), jnp.float32)),
        mesh=mesh,
        scratch_types=[
            pltpu.VMEM((2 * KT * R,), jnp.int32),
            pltpu.VMEM((2 * R, CH), jnp.float32),
            pltpu.VMEM((8, CH), jnp.float32),
            pltpu.SemaphoreType.DMA,
            pltpu.SemaphoreType.DMA,
            pltpu.SemaphoreType.DMA,
        ],
    )


def _tap_matmul(x, W, a=None, b=None):
    N, CIN = x.shape
    KP, _, COUT = W.shape
    TILE = 10000
    G = max(d for d in range(1, 5) if KP % d == 0)
    assert N % TILE == 0 and KP % G == 0
    T = N // TILE
    affine = a is not None

    def body(*refs):
        if affine:
            x_ref, w_ref, a_ref, b_ref, o_ref = refs
        else:
            x_ref, w_ref, o_ref = refs
        xv = x_ref[...]
        if affine:
            xv = jnp.maximum(xv * a_ref[...] + b_ref[...], 0.0)
        for g in range(G):
            o_ref[g] = jnp.dot(xv, w_ref[g], preferred_element_type=jnp.float32)

    in_specs = [
        pl.BlockSpec((TILE, CIN), lambda t, j: (t, 0)),
        pl.BlockSpec((G, CIN, COUT), lambda t, j: (j, 0, 0)),
    ]
    args = [x, W]
    if affine:
        in_specs += [
            pl.BlockSpec((1, CIN), lambda t, j: (0, 0)),
            pl.BlockSpec((1, CIN), lambda t, j: (0, 0)),
        ]
        args += [a, b]
    z3 = pl.pallas_call(
        body,
        grid=(T, KP // G),
        in_specs=in_specs,
        out_specs=pl.BlockSpec((G, TILE, COUT), lambda t, j: (j, t, 0)),
        out_shape=jax.ShapeDtypeStruct((KP, N, COUT), jnp.float32),
    )(*args)
    return z3.reshape(KP * N, COUT)


def _residual_epilogue(h, a, b, res):
    N, CH = h.shape
    TILE = 2000
    T = N // TILE

    def body(h_ref, a_ref, b_ref, r_ref, o_ref):
        o_ref[...] = jnp.maximum(
            h_ref[...] * a_ref[...] + b_ref[...] + r_ref[...], 0.0)

    return pl.pallas_call(
        body,
        grid=(T,),
        in_specs=[
            pl.BlockSpec((TILE, CH), lambda t: (t, 0)),
            pl.BlockSpec((1, CH), lambda t: (0, 0)),
            pl.BlockSpec((1, CH), lambda t: (0, 0)),
            pl.BlockSpec((TILE, CH), lambda t: (t, 0)),
        ],
        out_specs=pl.BlockSpec((TILE, CH), lambda t: (t, 0)),
        out_shape=jax.ShapeDtypeStruct((N, CH), jnp.float32),
    )(h, a, b, res)


def _bn_affine(stats, gamma, beta, n, eps=1e-5):
    mu = stats[0] / n
    var = stats[1] / n - mu * mu
    a = gamma * lax.rsqrt(var + eps)
    b = beta - mu * a
    return a.reshape(1, -1), b.reshape(1, -1)


def kernel(feats, W1, gamma1, beta1, W2, gamma2, beta2, in_idx, out_idx, valid):
    N, _ = feats.shape
    K = W1.shape[0]

    in_f = in_idx.reshape(-1).astype(jnp.int32)
    out_f = out_idx.reshape(-1).astype(jnp.int32)
    val_f = valid.reshape(-1).astype(jnp.float32)

    CH = feats.shape[1]
    pad = jnp.zeros((1,) + W1.shape[1:], W1.dtype)
    W1p = jnp.concatenate([W1, pad], axis=0)
    W2p = jnp.concatenate([W2, pad], axis=0)

    g = _build_invert(K, N, K)(out_f, in_f, val_f)
    gacc = _build_gather_accum(K, N, CH, 0, False)

    h1, p1 = gacc(_tap_matmul(feats, W1p), g)
    a1, b1 = _bn_affine(jnp.sum(p1, axis=0), gamma1, beta1, N)

    h2, p2 = gacc(_tap_matmul(h1, W2p, a1, b1), g)
    a2, b2 = _bn_affine(jnp.sum(p2, axis=0), gamma2, beta2, N)

    return _residual_epilogue(h2, a2, b2, feats)
